# Initial kernel scaffold; baseline (speedup 1.0000x reference)
import jax
import jax.numpy as jnp
from jax.experimental import pallas as pl


def kernel(q_len, k_len, table):
    raise NotImplementedError("write your pallas kernel here")


# speedup vs baseline: 41.1474x; 41.1474x over previous
import jax
import jax.numpy as jnp
from jax import lax
from jax.experimental import pallas as pl
from jax.experimental.pallas import tpu as pltpu
from jax.experimental.pallas import tpu_sc as plsc

H = 16
P = 2048
NREL = 2 * P - 1
SRC_LEN = NREL + 17
ROWS_PER_W = P // 2
STRIPES_PER_W = ROWS_PER_W // 8
LAG = 8


def _sc_body(src_hbm, out_hbm, src_v, sem):
    c = lax.axis_index("c")
    s = lax.axis_index("s")
    wid = s * 2 + c
    h = wid // 2
    i0w = (wid % 2) * ROWS_PER_W

    pltpu.sync_copy(src_hbm.at[h], src_v)

    def fire(b, carry):
        i0 = i0w + b * 8
        start = pl.multiple_of((P - 8) - i0, 8)
        pltpu.make_async_copy(
            src_v.at[:, pl.ds(start, P)],
            out_hbm.at[0, h, pl.ds(i0, 8), :],
            sem,
        ).start()

        @pl.when(b >= LAG)
        def _():
            pltpu.make_async_copy(
                src_v.at[:, pl.ds(0, P)],
                out_hbm.at[0, h, pl.ds(i0w, 8), :],
                sem,
            ).wait()

        return carry

    lax.fori_loop(0, STRIPES_PER_W, fire, 0)

    def drain(b, carry):
        pltpu.make_async_copy(
            src_v.at[:, pl.ds(0, P)],
            out_hbm.at[0, h, pl.ds(i0w, 8), :],
            sem,
        ).wait()
        return carry

    lax.fori_loop(0, LAG, drain, 0)


@jax.jit
def _rpb(table):
    rev = jnp.flip(table, axis=0).T
    rev = jnp.pad(rev, ((0, 0), (0, SRC_LEN + 7 - NREL)))
    src8 = jnp.stack([rev[:, 7 - r:7 - r + SRC_LEN] for r in range(8)], axis=1)

    mesh = plsc.VectorSubcoreMesh(core_axis_name="c", subcore_axis_name="s")
    out = pl.kernel(
        _sc_body,
        out_type=jax.ShapeDtypeStruct((1, H, P, P), jnp.float32),
        mesh=mesh,
        scratch_types=[
            pltpu.VMEM((8, SRC_LEN), jnp.float32),
            pltpu.SemaphoreType.DMA,
        ],
        compiler_params=pltpu.CompilerParams(use_tc_tiling_on_sc=False),
    )(src8)
    return out


def kernel(q_len, k_len, table):
    return _rpb(table)


# speedup vs baseline: 90.8987x; 2.2091x over previous
import jax
import jax.numpy as jnp
from jax import lax
from jax.experimental import pallas as pl
from jax.experimental.pallas import tpu as pltpu
from jax.experimental.pallas import tpu_sc as plsc

H = 16
P = 2048
NREL = 2 * P - 1
SRC_LEN = 3968
NG = 16
PAIRS_PER_W = NG // 2
K_PER_PAIR = 16


def _sc_body(src_hbm, out_hbm, buf, sem0, sem1):
    c = lax.axis_index("c")
    h = lax.axis_index("s")

    sems = (sem0, sem1)
    for p in range(PAIRS_PER_W):
        b = p % 2
        g = 8 * c + p

        if p >= 2:
            for _ in range(K_PER_PAIR):
                pltpu.make_async_copy(
                    buf.at[b, :, pl.ds(0, P)], out_hbm.at[0, h, pl.ds(0, 8), :], sems[b]
                ).wait()

        pltpu.sync_copy(src_hbm.at[h, g], buf.at[b])

        for k in range(K_PER_PAIR):
            i0 = pl.multiple_of(2040 - 8 * g - 128 * k, 8)
            pltpu.make_async_copy(
                buf.at[b, :, pl.ds(128 * k, P)],
                out_hbm.at[0, h, pl.ds(i0, 8), :],
                sems[b],
            ).start()

    for b in range(2):
        for _ in range(K_PER_PAIR):
            pltpu.make_async_copy(
                buf.at[b, :, pl.ds(0, P)], out_hbm.at[0, h, pl.ds(0, 8), :], sems[b]
            ).wait()


@jax.jit
def _rpb(table):
    rev = jnp.flip(table, axis=0).T
    src = jnp.stack(
        [
            jnp.stack(
                [rev[:, 8 * g + 7 - r:8 * g + 7 - r + SRC_LEN] for r in range(8)],
                axis=1,
            )
            for g in range(NG)
        ],
        axis=1,
    )

    mesh = plsc.VectorSubcoreMesh(core_axis_name="c", subcore_axis_name="s")
    out = pl.kernel(
        _sc_body,
        out_type=jax.ShapeDtypeStruct((1, H, P, P), jnp.float32),
        mesh=mesh,
        scratch_types=[
            pltpu.VMEM((2, 8, SRC_LEN), jnp.float32),
            pltpu.SemaphoreType.DMA,
            pltpu.SemaphoreType.DMA,
        ],
    )(src)
    return out


def kernel(q_len, k_len, table):
    return _rpb(table)


# speedup vs baseline: 97.0431x; 1.0676x over previous
import jax
import jax.numpy as jnp
from jax import lax
from jax.experimental import pallas as pl
from jax.experimental.pallas import tpu as pltpu
from jax.experimental.pallas import tpu_sc as plsc

H = 16
P = 2048
NREL = 2 * P - 1
SRC_LEN = 3968
VLEN = 4088
NG = 16
PAIRS_PER_W = NG // 2
K_PER_PAIR = 16
NBUF = 3


def _sc_body(src_hbm, out_hbm, buf, so0, so1, so2, si0, si1, si2):
    c = lax.axis_index("c")
    h = lax.axis_index("s")
    so = (so0, so1, so2)
    si = (si0, si1, si2)

    def in_copy(p):
        b = p % NBUF
        return pltpu.make_async_copy(src_hbm.at[h, 8 * c + p], buf.at[b], si[b])

    def drain_outs(b):
        for _ in range(K_PER_PAIR):
            pltpu.make_async_copy(
                buf.at[b, :, pl.ds(0, P)],
                out_hbm.at[0, h, pl.ds(0, 8), :],
                so[b],
            ).wait()

    in_copy(0).start()
    in_copy(1).start()
    for p in range(PAIRS_PER_W):
        b = p % NBUF
        in_copy(p).wait()
        for k in range(K_PER_PAIR):
            i0 = pl.multiple_of(2040 - 8 * (8 * c + p) - 128 * k, 8)
            pltpu.make_async_copy(
                buf.at[b, :, pl.ds(128 * k, P)],
                out_hbm.at[0, h, pl.ds(i0, 8), :],
                so[b],
            ).start()
        if p + 2 < PAIRS_PER_W:
            if p >= 1:
                drain_outs((p + 2) % NBUF)
            in_copy(p + 2).start()

    drain_outs((PAIRS_PER_W - 3) % NBUF)
    drain_outs((PAIRS_PER_W - 2) % NBUF)
    drain_outs((PAIRS_PER_W - 1) % NBUF)


@jax.jit
def _rpb(table):
    rev = jnp.flip(table, axis=0).T
    vflip = jnp.stack([rev[:, 7 - r:7 - r + VLEN] for r in range(8)], axis=1)
    src = jnp.stack([vflip[:, :, 8 * g:8 * g + SRC_LEN] for g in range(NG)],
                    axis=1)

    mesh = plsc.VectorSubcoreMesh(core_axis_name="c", subcore_axis_name="s")
    out = pl.kernel(
        _sc_body,
        out_type=jax.ShapeDtypeStruct((1, H, P, P), jnp.float32),
        mesh=mesh,
        scratch_types=[
            pltpu.VMEM((NBUF, 8, SRC_LEN), jnp.float32),
            pltpu.SemaphoreType.DMA,
            pltpu.SemaphoreType.DMA,
            pltpu.SemaphoreType.DMA,
            pltpu.SemaphoreType.DMA,
            pltpu.SemaphoreType.DMA,
            pltpu.SemaphoreType.DMA,
        ],
    )(src)
    return out


def kernel(q_len, k_len, table):
    return _rpb(table)


# speedup vs baseline: 102.7145x; 1.0584x over previous
import jax
import jax.numpy as jnp
from jax import lax
from jax.experimental import pallas as pl
from jax.experimental.pallas import tpu as pltpu
from jax.experimental.pallas import tpu_sc as plsc

H = 16
P = 2048
NREL = 2 * P - 1
SRC_LEN = 3968
VLEN = 4088
NG = 16
PAIRS_PER_W = NG // 2
K_PER_PAIR = 16
NBUF = 3


def _sc_body(src_hbm, out_hbm, buf, so0, so1, so2, si0, si1, si2):
    c = lax.axis_index("c")
    h = lax.axis_index("s")
    so = (so0, so1, so2)
    si = (si0, si1, si2)

    def in_copy(p):
        b = p % NBUF
        return pltpu.make_async_copy(src_hbm.at[h, 8 * c + p], buf.at[b], si[b])

    def drain_outs(b):
        for _ in range(K_PER_PAIR):
            pltpu.make_async_copy(
                buf.at[b, :, pl.ds(0, P)],
                out_hbm.at[0, h, pl.ds(0, 8), :],
                so[b],
            ).wait()

    in_copy(0).start()
    in_copy(1).start()
    for p in range(PAIRS_PER_W):
        b = p % NBUF
        in_copy(p).wait()
        for k in range(K_PER_PAIR):
            i0 = pl.multiple_of(2040 - 8 * (8 * c + p) - 128 * k, 8)
            pltpu.make_async_copy(
                buf.at[b, :, pl.ds(128 * k, P)],
                out_hbm.at[0, h, pl.ds(i0, 8), :],
                so[b],
            ).start()
        if p + 2 < PAIRS_PER_W:
            if p >= 1:
                drain_outs((p + 2) % NBUF)
            in_copy(p + 2).start()

    drain_outs((PAIRS_PER_W - 3) % NBUF)
    drain_outs((PAIRS_PER_W - 2) % NBUF)
    drain_outs((PAIRS_PER_W - 1) % NBUF)


@jax.jit
def _rpb(table):
    rev = jnp.flip(jnp.pad(table, ((1, 0), (0, 0))), axis=0).T
    vflip = jnp.stack([rev[:, 7 - r:7 - r + VLEN] for r in range(8)], axis=1)
    src = jnp.stack([vflip[:, :, 8 * g:8 * g + SRC_LEN] for g in range(NG)],
                    axis=1)

    mesh = plsc.VectorSubcoreMesh(core_axis_name="c", subcore_axis_name="s")
    out = pl.kernel(
        _sc_body,
        out_type=jax.ShapeDtypeStruct((1, H, P, P), jnp.float32),
        mesh=mesh,
        scratch_types=[
            pltpu.VMEM((NBUF, 8, SRC_LEN), jnp.float32),
            pltpu.SemaphoreType.DMA,
            pltpu.SemaphoreType.DMA,
            pltpu.SemaphoreType.DMA,
            pltpu.SemaphoreType.DMA,
            pltpu.SemaphoreType.DMA,
            pltpu.SemaphoreType.DMA,
        ],
    )(src)
    return out


def kernel(q_len, k_len, table):
    return _rpb(table)
